# baseline (speedup 1.0000x reference)
import functools

import jax
import jax.numpy as jnp
from jax import lax
from jax.experimental import pallas as pl
from jax.experimental.pallas import tpu as pltpu
from jax.experimental.pallas import tpu_sc as plsc

_N = 8192
_D = 256
_BR = 1024
_BC = 1024
_NR = _N // _BR
_NC = _N // _BC
_BETA = 0.25


def _argmin_body(zf_ref, wt_ref, zn_ref, wn_ref, idx_ref, loss_ref,
                 bv_ref, bi_ref):
    i = pl.program_id(0)
    c = pl.program_id(1)
    s = lax.dot_general(zf_ref[...], wt_ref[...], (((1,), (0,)), ((), ())),
                        preferred_element_type=jnp.float32)
    d = (zn_ref[...] + wn_ref[...]) - 2.0 * s
    m = jnp.min(d, axis=1, keepdims=True)
    ids = lax.broadcasted_iota(jnp.int32, (_BR, _BC), 1) + c * _BC
    a = jnp.min(jnp.where(d == m, ids, jnp.int32(2 ** 30)), axis=1,
                keepdims=True)

    @pl.when(c == 0)
    def _():
        bv_ref[...] = m
        bi_ref[...] = a

    @pl.when(c > 0)
    def _():
        upd = m < bv_ref[...]
        bv_ref[...] = jnp.where(upd, m, bv_ref[...])
        bi_ref[...] = jnp.where(upd, a, bi_ref[...])

    @pl.when(c == _NC - 1)
    def _():
        idx_ref[...] = bi_ref[...]
        part = jnp.sum(bv_ref[...])

        @pl.when(i == 0)
        def _():
            loss_ref[0, 0] = part

        @pl.when(i > 0)
        def _():
            loss_ref[0, 0] = loss_ref[0, 0] + part

        @pl.when(i == _NR - 1)
        def _():
            loss_ref[0, 0] = loss_ref[0, 0] * ((1.0 + _BETA) / (_N * _D))


def _onehot_body(idx_ref, enc_ref, perp_ref, acc_ref, ent_ref):
    c = pl.program_id(0)
    b = pl.program_id(1)
    ids = lax.broadcasted_iota(jnp.int32, (_BR, _BC), 1) + c * _BC
    oh = (idx_ref[...] == ids).astype(jnp.float32)
    enc_ref[...] = oh
    col = jnp.sum(oh, axis=0, keepdims=True)

    @pl.when(b == 0)
    def _():
        acc_ref[...] = col

    @pl.when(b > 0)
    def _():
        acc_ref[...] = acc_ref[...] + col

    @pl.when(b == _NR - 1)
    def _():
        p = acc_ref[...] * (1.0 / _N)
        t = jnp.sum(p * jnp.log(p + 1e-10))

        @pl.when(c == 0)
        def _():
            ent_ref[0, 0] = t

        @pl.when(c > 0)
        def _():
            ent_ref[0, 0] = ent_ref[0, 0] + t

        @pl.when(c == _NC - 1)
        def _():
            perp_ref[0, 0] = jnp.exp(-ent_ref[0, 0])


def _make_sc_gather(num_cores, num_subcores):
    nw = num_cores * num_subcores
    bpw = _N // nw
    mesh = plsc.VectorSubcoreMesh(core_axis_name="c", subcore_axis_name="s")

    @functools.partial(
        pl.kernel, mesh=mesh,
        out_type=jax.ShapeDtypeStruct((_N, _D), jnp.float32),
        scratch_types=[
            pltpu.VMEM((bpw,), jnp.int32),
            pltpu.VMEM((bpw, _D), jnp.float32),
            pltpu.SemaphoreType.DMA,
        ],
    )
    def gather(table_hbm, idx_hbm, out_hbm, idx_v, rows_v, sem):
        wid = lax.axis_index("s") * num_cores + lax.axis_index("c")
        base = wid * bpw
        pltpu.sync_copy(idx_hbm.at[pl.ds(base, bpw)], idx_v)
        pltpu.async_copy(table_hbm.at[idx_v], rows_v, sem).wait()
        pltpu.sync_copy(rows_v, out_hbm.at[pl.ds(base, bpw)])

    return gather


_argmin_call = pl.pallas_call(
    _argmin_body,
    grid=(_NR, _NC),
    in_specs=[
        pl.BlockSpec((_BR, _D), lambda i, c: (i, 0)),
        pl.BlockSpec((_D, _BC), lambda i, c: (0, c)),
        pl.BlockSpec((_BR, 1), lambda i, c: (i, 0)),
        pl.BlockSpec((1, _BC), lambda i, c: (0, c)),
    ],
    out_specs=[
        pl.BlockSpec((_BR, 1), lambda i, c: (i, 0)),
        pl.BlockSpec(memory_space=pltpu.SMEM),
    ],
    out_shape=[
        jax.ShapeDtypeStruct((_N, 1), jnp.int32),
        jax.ShapeDtypeStruct((1, 1), jnp.float32),
    ],
    scratch_shapes=[
        pltpu.VMEM((_BR, 1), jnp.float32),
        pltpu.VMEM((_BR, 1), jnp.int32),
    ],
    compiler_params=pltpu.CompilerParams(
        dimension_semantics=("arbitrary", "arbitrary")),
)

_onehot_call = pl.pallas_call(
    _onehot_body,
    grid=(_NC, _NR),
    in_specs=[
        pl.BlockSpec((_BR, 1), lambda c, b: (b, 0)),
    ],
    out_specs=[
        pl.BlockSpec((_BR, _BC), lambda c, b: (b, c)),
        pl.BlockSpec(memory_space=pltpu.SMEM),
    ],
    out_shape=[
        jax.ShapeDtypeStruct((_N, _N), jnp.float32),
        jax.ShapeDtypeStruct((1, 1), jnp.float32),
    ],
    scratch_shapes=[
        pltpu.VMEM((1, _BC), jnp.float32),
        pltpu.SMEM((1, 1), jnp.float32),
    ],
    compiler_params=pltpu.CompilerParams(
        dimension_semantics=("arbitrary", "arbitrary")),
)


def kernel(z, W):
    b, cdim, h, w = z.shape
    zt = jnp.transpose(z, (0, 2, 3, 1))
    zf = zt.reshape(-1, _D)
    znorm = jnp.sum(zf ** 2, axis=1, keepdims=True)
    wnorm = jnp.sum(W ** 2, axis=1).reshape(1, _N)
    idx2, loss = _argmin_call(zf, W.T, znorm, wnorm)
    enc, perp = _onehot_call(idx2)
    idx = idx2.reshape(_N)

    info = plsc.get_sparse_core_info()
    zq = _make_sc_gather(info.num_cores, info.num_subcores)(W, idx)
    z_q_out = jnp.transpose(zq.reshape(b, h, w, cdim), (0, 3, 1, 2))
    return (z_q_out, loss[0, 0], perp[0, 0], enc, idx)


# speedup vs baseline: 1.2324x; 1.2324x over previous
import functools

import jax
import jax.numpy as jnp
from jax import lax
from jax.experimental import pallas as pl
from jax.experimental.pallas import tpu as pltpu
from jax.experimental.pallas import tpu_sc as plsc

_N = 8192
_D = 256
_BR = 1024
_BC = 1024
_NR = _N // _BR
_NC = _N // _BC
_BETA = 0.25


def _vq_body(zf_ref, wt2_ref, zn_ref, wn_ref,
             idx_ref, enc_ref, loss_ref, perp_ref,
             bv_ref, bi_ref, bip_ref, cnt_ref, ent_ref):
    i = pl.program_id(0)
    c = pl.program_id(1)
    idsi = lax.broadcasted_iota(jnp.int32, (_BR, _BC), 1)
    idsf = idsi.astype(jnp.float32)
    off = pl.multiple_of(c * _BC, _BC)

    @pl.when(c == 0)
    def _snap():
        bip_ref[...] = bi_ref[...]

    @pl.when(i < _NR)
    def _compute():
        s2 = lax.dot_general(zf_ref[...], wt2_ref[:, pl.ds(off, _BC)],
                             (((1,), (0,)), ((), ())),
                             preferred_element_type=jnp.float32)
        d = (zn_ref[...] + wn_ref[0:1, pl.ds(off, _BC)]) + s2
        m = jnp.min(d, axis=1, keepdims=True)
        a_loc = jnp.min(jnp.where(d == m, idsf, jnp.float32(3e38)), axis=1,
                        keepdims=True)
        a = a_loc.astype(jnp.int32) + c * _BC

        @pl.when(c == 0)
        def _():
            bv_ref[...] = m
            bi_ref[...] = a

        @pl.when(c > 0)
        def _():
            upd = m < bv_ref[...]
            bv_ref[...] = jnp.where(upd, m, bv_ref[...])
            bi_ref[...] = jnp.where(upd, a, bi_ref[...])

        @pl.when(c == _NC - 1)
        def _():
            idx_ref[...] = bi_ref[...]
            part = jnp.sum(bv_ref[...])

            @pl.when(i == 0)
            def _():
                loss_ref[0, 0] = part

            @pl.when(i > 0)
            def _():
                loss_ref[0, 0] = loss_ref[0, 0] + part

    @pl.when(i > 0)
    def _emit():
        oh = ((bip_ref[...] - c * _BC) == idsi).astype(jnp.float32)
        enc_ref[...] = oh
        col = jnp.sum(oh, axis=0, keepdims=True)

        @pl.when(i == 1)
        def _():
            cnt_ref[0:1, pl.ds(off, _BC)] = col

        @pl.when(i > 1)
        def _():
            cnt_ref[0:1, pl.ds(off, _BC)] = (
                cnt_ref[0:1, pl.ds(off, _BC)] + col)

    @pl.when(i == _NR)
    def _final():
        p = cnt_ref[0:1, pl.ds(off, _BC)] * (1.0 / _N)
        tt = jnp.sum(p * jnp.log(p + 1e-10))

        @pl.when(c == 0)
        def _():
            ent_ref[0, 0] = tt

        @pl.when(c > 0)
        def _():
            ent_ref[0, 0] = ent_ref[0, 0] + tt

        @pl.when(c == _NC - 1)
        def _():
            perp_ref[0, 0] = jnp.exp(-ent_ref[0, 0])
            loss_ref[0, 0] = loss_ref[0, 0] * ((1.0 + _BETA) / (_N * _D))


def _make_sc_gather(num_cores, num_subcores):
    nw = num_cores * num_subcores
    bpw = _N // nw
    mesh = plsc.VectorSubcoreMesh(core_axis_name="c", subcore_axis_name="s")

    @functools.partial(
        pl.kernel, mesh=mesh,
        out_type=jax.ShapeDtypeStruct((_N, _D), jnp.float32),
        scratch_types=[
            pltpu.VMEM((bpw,), jnp.int32),
            pltpu.VMEM((bpw, _D), jnp.float32),
            pltpu.SemaphoreType.DMA,
        ],
    )
    def gather(table_hbm, idx_hbm, out_hbm, idx_v, rows_v, sem):
        wid = lax.axis_index("s") * num_cores + lax.axis_index("c")
        base = wid * bpw
        pltpu.sync_copy(idx_hbm.at[pl.ds(base, bpw)], idx_v)
        pltpu.async_copy(table_hbm.at[idx_v], rows_v, sem).wait()
        pltpu.sync_copy(rows_v, out_hbm.at[pl.ds(base, bpw)])

    return gather


_vq_call = pl.pallas_call(
    _vq_body,
    grid=(_NR + 1, _NC),
    in_specs=[
        pl.BlockSpec((_BR, _D), lambda i, c: (jnp.minimum(i, _NR - 1), 0)),
        pl.BlockSpec((_D, _N), lambda i, c: (0, 0)),
        pl.BlockSpec((_BR, 1), lambda i, c: (jnp.minimum(i, _NR - 1), 0)),
        pl.BlockSpec((1, _N), lambda i, c: (0, 0)),
    ],
    out_specs=[
        pl.BlockSpec((_BR, 1), lambda i, c: (jnp.minimum(i, _NR - 1), 0)),
        pl.BlockSpec((_BR, _BC), lambda i, c: (jnp.maximum(i - 1, 0), c)),
        pl.BlockSpec(memory_space=pltpu.SMEM),
        pl.BlockSpec(memory_space=pltpu.SMEM),
    ],
    out_shape=[
        jax.ShapeDtypeStruct((_N, 1), jnp.int32),
        jax.ShapeDtypeStruct((_N, _N), jnp.float32),
        jax.ShapeDtypeStruct((1, 1), jnp.float32),
        jax.ShapeDtypeStruct((1, 1), jnp.float32),
    ],
    scratch_shapes=[
        pltpu.VMEM((_BR, 1), jnp.float32),
        pltpu.VMEM((_BR, 1), jnp.int32),
        pltpu.VMEM((_BR, 1), jnp.int32),
        pltpu.VMEM((1, _N), jnp.float32),
        pltpu.SMEM((1, 1), jnp.float32),
    ],
    compiler_params=pltpu.CompilerParams(
        dimension_semantics=("arbitrary", "arbitrary")),
)


def kernel(z, W):
    b, cdim, h, w = z.shape
    zt = jnp.transpose(z, (0, 2, 3, 1))
    zf = zt.reshape(-1, _D)
    znorm = jnp.sum(zf ** 2, axis=1, keepdims=True)
    wnorm = jnp.sum(W ** 2, axis=1).reshape(1, _N)
    wt2 = (-2.0 * W).T
    idx2, enc, loss, perp = _vq_call(zf, wt2, znorm, wnorm)
    idx = idx2.reshape(_N)

    info = plsc.get_sparse_core_info()
    zq = _make_sc_gather(info.num_cores, info.num_subcores)(W, idx)
    z_q_out = jnp.transpose(zq.reshape(b, h, w, cdim), (0, 3, 1, 2))
    return (z_q_out, loss[0, 0], perp[0, 0], enc, idx)


# speedup vs baseline: 1.3279x; 1.0775x over previous
import functools

import jax
import jax.numpy as jnp
from jax import lax
from jax.experimental import pallas as pl
from jax.experimental.pallas import tpu as pltpu
from jax.experimental.pallas import tpu_sc as plsc

_N = 8192
_D = 256
_BR = 1024
_BC = 1024
_NR = _N // _BR
_NC = _N // _BC
_BETA = 0.25


def _vq_body(zf_ref, wt2_ref, zn_ref, wn_ref,
             idx_ref, enc_ref, loss_ref, perp_ref,
             bv_ref, bi_ref, bip_ref, cnt_ref, ent_ref):
    i = pl.program_id(0)
    c = pl.program_id(1)
    idsi = lax.broadcasted_iota(jnp.int32, (1, _BC), 1)
    lane = lax.broadcasted_iota(jnp.int32, (1, 128), 1).astype(jnp.float32)
    off = pl.multiple_of(c * _BC, _BC)

    @pl.when(c == 0)
    def _snap():
        bip_ref[...] = bi_ref[...]

    @pl.when(i < _NR)
    def _compute():
        s2 = lax.dot_general(zf_ref[...], wt2_ref[:, pl.ds(off, _BC)],
                             (((1,), (0,)), ((), ())),
                             preferred_element_type=jnp.float32)
        zn = zn_ref[...]
        m = None
        av = None
        for k in range(_BC // 128):
            dk = ((zn + wn_ref[0:1, pl.ds(off + k * 128, 128)])
                  + s2[:, k * 128:(k + 1) * 128])
            colv = lane + jnp.float32(k * 128)
            if m is None:
                m = dk
                av = jnp.broadcast_to(colv, dk.shape)
            else:
                upd = dk < m
                m = jnp.where(upd, dk, m)
                av = jnp.where(upd, colv, av)
        mrow = jnp.min(m, axis=1, keepdims=True)
        a_loc = jnp.min(jnp.where(m == mrow, av, jnp.float32(3e38)),
                        axis=1, keepdims=True)
        a = a_loc.astype(jnp.int32) + c * _BC
        m = mrow

        @pl.when(c == 0)
        def _():
            bv_ref[...] = m
            bi_ref[...] = a

        @pl.when(c > 0)
        def _():
            upd = m < bv_ref[...]
            bv_ref[...] = jnp.where(upd, m, bv_ref[...])
            bi_ref[...] = jnp.where(upd, a, bi_ref[...])

        @pl.when(c == _NC - 1)
        def _():
            idx_ref[...] = bi_ref[...]
            part = jnp.sum(bv_ref[...])

            @pl.when(i == 0)
            def _():
                loss_ref[0, 0] = part

            @pl.when(i > 0)
            def _():
                loss_ref[0, 0] = loss_ref[0, 0] + part

    @pl.when(i > 0)
    def _emit():
        oh = ((bip_ref[...] - c * _BC) == idsi).astype(jnp.float32)
        enc_ref[...] = oh
        col = jnp.sum(oh, axis=0, keepdims=True)

        @pl.when(i == 1)
        def _():
            cnt_ref[0:1, pl.ds(off, _BC)] = col

        @pl.when(i > 1)
        def _():
            cnt_ref[0:1, pl.ds(off, _BC)] = (
                cnt_ref[0:1, pl.ds(off, _BC)] + col)

    @pl.when(i == _NR)
    def _final():
        p = cnt_ref[0:1, pl.ds(off, _BC)] * (1.0 / _N)
        tt = jnp.sum(p * jnp.log(p + 1e-10))

        @pl.when(c == 0)
        def _():
            ent_ref[0, 0] = tt

        @pl.when(c > 0)
        def _():
            ent_ref[0, 0] = ent_ref[0, 0] + tt

        @pl.when(c == _NC - 1)
        def _():
            perp_ref[0, 0] = jnp.exp(-ent_ref[0, 0])
            loss_ref[0, 0] = loss_ref[0, 0] * ((1.0 + _BETA) / (_N * _D))


def _make_sc_gather(num_cores, num_subcores):
    nw = num_cores * num_subcores
    bpw = _N // nw
    mesh = plsc.VectorSubcoreMesh(core_axis_name="c", subcore_axis_name="s")

    @functools.partial(
        pl.kernel, mesh=mesh,
        out_type=jax.ShapeDtypeStruct((_N, _D), jnp.float32),
        scratch_types=[
            pltpu.VMEM((bpw,), jnp.int32),
            pltpu.VMEM((bpw, _D), jnp.float32),
            pltpu.SemaphoreType.DMA,
        ],
    )
    def gather(table_hbm, idx_hbm, out_hbm, idx_v, rows_v, sem):
        wid = lax.axis_index("s") * num_cores + lax.axis_index("c")
        base = wid * bpw
        pltpu.sync_copy(idx_hbm.at[pl.ds(base, bpw)], idx_v)
        pltpu.async_copy(table_hbm.at[idx_v], rows_v, sem).wait()
        pltpu.sync_copy(rows_v, out_hbm.at[pl.ds(base, bpw)])

    return gather


_vq_call = pl.pallas_call(
    _vq_body,
    grid=(_NR + 1, _NC),
    in_specs=[
        pl.BlockSpec((_BR, _D), lambda i, c: (jnp.minimum(i, _NR - 1), 0)),
        pl.BlockSpec((_D, _N), lambda i, c: (0, 0)),
        pl.BlockSpec((_BR, 1), lambda i, c: (jnp.minimum(i, _NR - 1), 0)),
        pl.BlockSpec((1, _N), lambda i, c: (0, 0)),
    ],
    out_specs=[
        pl.BlockSpec((_BR, 1), lambda i, c: (jnp.minimum(i, _NR - 1), 0)),
        pl.BlockSpec((_BR, _BC), lambda i, c: (jnp.maximum(i - 1, 0), c)),
        pl.BlockSpec(memory_space=pltpu.SMEM),
        pl.BlockSpec(memory_space=pltpu.SMEM),
    ],
    out_shape=[
        jax.ShapeDtypeStruct((_N, 1), jnp.int32),
        jax.ShapeDtypeStruct((_N, _N), jnp.float32),
        jax.ShapeDtypeStruct((1, 1), jnp.float32),
        jax.ShapeDtypeStruct((1, 1), jnp.float32),
    ],
    scratch_shapes=[
        pltpu.VMEM((_BR, 1), jnp.float32),
        pltpu.VMEM((_BR, 1), jnp.int32),
        pltpu.VMEM((_BR, 1), jnp.int32),
        pltpu.VMEM((1, _N), jnp.float32),
        pltpu.SMEM((1, 1), jnp.float32),
    ],
    compiler_params=pltpu.CompilerParams(
        dimension_semantics=("arbitrary", "arbitrary")),
)


def kernel(z, W):
    b, cdim, h, w = z.shape
    zt = jnp.transpose(z, (0, 2, 3, 1))
    zf = zt.reshape(-1, _D)
    znorm = jnp.sum(zf ** 2, axis=1, keepdims=True)
    wnorm = jnp.sum(W ** 2, axis=1).reshape(1, _N)
    wt2 = (-2.0 * W).T
    idx2, enc, loss, perp = _vq_call(zf, wt2, znorm, wnorm)
    idx = idx2.reshape(_N)

    info = plsc.get_sparse_core_info()
    zq = _make_sc_gather(info.num_cores, info.num_subcores)(W, idx)
    z_q_out = jnp.transpose(zq.reshape(b, h, w, cdim), (0, 3, 1, 2))
    return (z_q_out, loss[0, 0], perp[0, 0], enc, idx)


# speedup vs baseline: 1.6419x; 1.2365x over previous
import functools

import jax
import jax.numpy as jnp
from jax import lax
from jax.experimental import pallas as pl
from jax.experimental.pallas import tpu as pltpu
from jax.experimental.pallas import tpu_sc as plsc

_N = 8192
_D = 256
_BR = 512
_BC = 8192
_NR = _N // _BR
_NC = _N // _BC
_BETA = 0.25


def _vq_body(zf_ref, wt2_ref, zn_ref, wn_ref,
             idx_ref, enc_ref, loss_ref, perp_ref,
             bv_ref, bi_ref, bip_ref, cnt_ref, ent_ref):
    i = pl.program_id(0)
    c = pl.program_id(1)
    idsi = lax.broadcasted_iota(jnp.int32, (1, _BC), 1)
    lane = lax.broadcasted_iota(jnp.int32, (1, 128), 1).astype(jnp.float32)
    off = pl.multiple_of(c * _BC, _BC)

    @pl.when(c == 0)
    def _snap():
        bip_ref[...] = bi_ref[...]

    @pl.when(i < _NR)
    def _compute():
        s2 = lax.dot_general(zf_ref[...], wt2_ref[:, pl.ds(off, _BC)],
                             (((1,), (0,)), ((), ())),
                             preferred_element_type=jnp.float32)
        zn = zn_ref[...]
        m = None
        av = None
        for k in range(_BC // 128):
            dk = ((zn + wn_ref[0:1, pl.ds(off + k * 128, 128)])
                  + s2[:, k * 128:(k + 1) * 128])
            colv = lane + jnp.float32(k * 128)
            if m is None:
                m = dk
                av = jnp.broadcast_to(colv, dk.shape)
            else:
                upd = dk < m
                m = jnp.where(upd, dk, m)
                av = jnp.where(upd, colv, av)
        mrow = jnp.min(m, axis=1, keepdims=True)
        a_loc = jnp.min(jnp.where(m == mrow, av, jnp.float32(3e38)),
                        axis=1, keepdims=True)
        a = a_loc.astype(jnp.int32) + c * _BC
        m = mrow

        @pl.when(c == 0)
        def _():
            bv_ref[...] = m
            bi_ref[...] = a

        @pl.when(c > 0)
        def _():
            upd = m < bv_ref[...]
            bv_ref[...] = jnp.where(upd, m, bv_ref[...])
            bi_ref[...] = jnp.where(upd, a, bi_ref[...])

        @pl.when(c == _NC - 1)
        def _():
            idx_ref[...] = bi_ref[...]
            part = jnp.sum(bv_ref[...])

            @pl.when(i == 0)
            def _():
                loss_ref[0, 0] = part

            @pl.when(i > 0)
            def _():
                loss_ref[0, 0] = loss_ref[0, 0] + part

    @pl.when(i > 0)
    def _emit():
        oh = ((bip_ref[...] - c * _BC) == idsi).astype(jnp.float32)
        enc_ref[...] = oh
        col = jnp.sum(oh, axis=0, keepdims=True)

        @pl.when(i == 1)
        def _():
            cnt_ref[0:1, pl.ds(off, _BC)] = col

        @pl.when(i > 1)
        def _():
            cnt_ref[0:1, pl.ds(off, _BC)] = (
                cnt_ref[0:1, pl.ds(off, _BC)] + col)

    @pl.when(i == _NR)
    def _final():
        p = cnt_ref[0:1, pl.ds(off, _BC)] * (1.0 / _N)
        tt = jnp.sum(p * jnp.log(p + 1e-10))

        @pl.when(c == 0)
        def _():
            ent_ref[0, 0] = tt

        @pl.when(c > 0)
        def _():
            ent_ref[0, 0] = ent_ref[0, 0] + tt

        @pl.when(c == _NC - 1)
        def _():
            perp_ref[0, 0] = jnp.exp(-ent_ref[0, 0])
            loss_ref[0, 0] = loss_ref[0, 0] * ((1.0 + _BETA) / (_N * _D))


def _make_sc_gather(num_cores, num_subcores):
    nw = num_cores * num_subcores
    bpw = _N // nw
    mesh = plsc.VectorSubcoreMesh(core_axis_name="c", subcore_axis_name="s")

    @functools.partial(
        pl.kernel, mesh=mesh,
        out_type=jax.ShapeDtypeStruct((_N, _D), jnp.float32),
        scratch_types=[
            pltpu.VMEM((bpw,), jnp.int32),
            pltpu.VMEM((bpw, _D), jnp.float32),
            pltpu.SemaphoreType.DMA,
        ],
    )
    def gather(table_hbm, idx_hbm, out_hbm, idx_v, rows_v, sem):
        wid = lax.axis_index("s") * num_cores + lax.axis_index("c")
        base = wid * bpw
        pltpu.sync_copy(idx_hbm.at[pl.ds(base, bpw)], idx_v)
        pltpu.async_copy(table_hbm.at[idx_v], rows_v, sem).wait()
        pltpu.sync_copy(rows_v, out_hbm.at[pl.ds(base, bpw)])

    return gather


_vq_call = pl.pallas_call(
    _vq_body,
    grid=(_NR + 1, _NC),
    in_specs=[
        pl.BlockSpec((_BR, _D), lambda i, c: (jnp.minimum(i, _NR - 1), 0)),
        pl.BlockSpec((_D, _N), lambda i, c: (0, 0)),
        pl.BlockSpec((_BR, 1), lambda i, c: (jnp.minimum(i, _NR - 1), 0)),
        pl.BlockSpec((1, _N), lambda i, c: (0, 0)),
    ],
    out_specs=[
        pl.BlockSpec((_BR, 1), lambda i, c: (jnp.minimum(i, _NR - 1), 0)),
        pl.BlockSpec((_BR, _BC), lambda i, c: (jnp.maximum(i - 1, 0), c)),
        pl.BlockSpec(memory_space=pltpu.SMEM),
        pl.BlockSpec(memory_space=pltpu.SMEM),
    ],
    out_shape=[
        jax.ShapeDtypeStruct((_N, 1), jnp.int32),
        jax.ShapeDtypeStruct((_N, _N), jnp.float32),
        jax.ShapeDtypeStruct((1, 1), jnp.float32),
        jax.ShapeDtypeStruct((1, 1), jnp.float32),
    ],
    scratch_shapes=[
        pltpu.VMEM((_BR, 1), jnp.float32),
        pltpu.VMEM((_BR, 1), jnp.int32),
        pltpu.VMEM((_BR, 1), jnp.int32),
        pltpu.VMEM((1, _N), jnp.float32),
        pltpu.SMEM((1, 1), jnp.float32),
    ],
    compiler_params=pltpu.CompilerParams(
        dimension_semantics=("arbitrary", "arbitrary")),
)


def kernel(z, W):
    b, cdim, h, w = z.shape
    zt = jnp.transpose(z, (0, 2, 3, 1))
    zf = zt.reshape(-1, _D)
    znorm = jnp.sum(zf ** 2, axis=1, keepdims=True)
    wnorm = jnp.sum(W ** 2, axis=1).reshape(1, _N)
    wt2 = (-2.0 * W).T
    idx2, enc, loss, perp = _vq_call(zf, wt2, znorm, wnorm)
    idx = idx2.reshape(_N)

    info = plsc.get_sparse_core_info()
    zq = _make_sc_gather(info.num_cores, info.num_subcores)(W, idx)
    z_q_out = jnp.transpose(zq.reshape(b, h, w, cdim), (0, 3, 1, 2))
    return (z_q_out, loss[0, 0], perp[0, 0], enc, idx)
